# baseline (speedup 1.0000x reference)
import functools
import math

import jax
import jax.numpy as jnp
import numpy as np
from jax import lax
from jax.experimental import pallas as pl
from jax.experimental.pallas import tpu as pltpu
from jax.experimental.pallas import tpu_sc as plsc

_B, _S = 1024, 512
_D, _T = 128, 16
_MAX_LEN = 512
_PAD_IDX = 1
_N = _B * _S


def _make_pe(max_len, d):
    position = np.arange(max_len, dtype=np.float32)[:, None]
    div_term = np.exp(
        np.arange(0, d, 2, dtype=np.float32) * -(math.log(10000.0) / d)
    )
    pe = np.zeros((max_len, d), dtype=np.float32)
    pe[:, 0::2] = np.sin(position * div_term)
    pe[:, 1::2] = np.cos(position * div_term)
    return pe


_SC_CHUNK = 128
_SC_NBUF = 4


def _sc_gather_fn(V, D, N, dtype=jnp.float32):
    info = plsc.get_sparse_core_info()
    nc, ns = info.num_cores, info.num_subcores
    nw = nc * ns
    n_per_w = N // nw
    n_super = n_per_w // (_SC_CHUNK * _SC_NBUF)
    assert n_per_w % (_SC_CHUNK * _SC_NBUF) == 0

    mesh = plsc.VectorSubcoreMesh(core_axis_name="c", subcore_axis_name="s")

    @functools.partial(
        pl.kernel,
        mesh=mesh,
        compiler_params=pltpu.CompilerParams(use_tc_tiling_on_sc=False),
        out_type=jax.ShapeDtypeStruct((N, D), dtype),
        scratch_types=[
            pltpu.VMEM((n_per_w,), jnp.int32),
            pltpu.VMEM((_SC_NBUF, _SC_CHUNK, D), dtype),
            pltpu.SemaphoreType.DMA,
            pltpu.SemaphoreType.DMA,
        ],
    )
    def k(table_hbm, idx_hbm, out_hbm, idx_v, rows_v, sem_g, sem_w):
        wid = lax.axis_index("s") * nc + lax.axis_index("c")
        base = wid * n_per_w
        pltpu.sync_copy(idx_hbm.at[pl.ds(base, n_per_w)], idx_v)

        def super_step(q, _):
            c0 = q * _SC_NBUF
            gathers = []
            for b in range(_SC_NBUF):
                gathers.append(pltpu.async_copy(
                    table_hbm.at[
                        idx_v.at[pl.ds((c0 + b) * _SC_CHUNK, _SC_CHUNK)]
                    ],
                    rows_v.at[b],
                    sem_g,
                ))
            for h in gathers:
                h.wait()
            writes = []
            for b in range(_SC_NBUF):
                writes.append(pltpu.async_copy(
                    rows_v.at[b],
                    out_hbm.at[pl.ds(base + (c0 + b) * _SC_CHUNK, _SC_CHUNK)],
                    sem_w,
                ))
            for h in writes:
                h.wait()
            return 0

        lax.fori_loop(0, n_super, super_step, 0, unroll=False)

    return k


_TOK = 8192


def _fast_sin(x):
    n = jnp.round(x * 0.15915494309189535)
    r = (x - n * 6.28125) - n * 0.0019353071795864769
    r2 = r * r
    p = 2.1470496156333067e-06
    p = p * r2 - 0.00019263169952706073
    p = p * r2 + 0.008308849931229644
    p = p * r2 - 0.16662401538302815
    p = p * r2 + 0.9999791148945345
    return p * r


def _epilogue_body(
    conc_ref,
    ints_ref,
    tsd_ref,
    w_ref,
    b_ref,
    gamma_ref,
    beta_ref,
    twc_ref,
    tphc_ref,
    awc_ref,
    aphc_ref,
    tv_ref,
    pe_ref,
    pepat_ref,
    pe0_ref,
    pe511_ref,
    out_ref,
):
    f32 = jnp.float32
    bf16 = jnp.bfloat16

    ints = ints_ref[...]
    vo, vof = ints[:, 0:1], ints[:, 1:2]
    cid, cidf = ints[:, 2:3], ints[:, 3:4]

    tri = tsd_ref[...]
    dd = (tri[0:1, :] - tri[1:2, :]).astype(f32)
    agr = tri[2:3, :].astype(f32)
    tfT = _fast_sin(dd * twc_ref[...] + tphc_ref[...])
    afT = _fast_sin(agr * awc_ref[...] + aphc_ref[...])
    taT = jnp.concatenate([tfT, afT], axis=0)

    w = w_ref[...]
    ci = conc_ref[...]
    e_lo = lax.bitcast_convert_type(ci << 16, f32)
    e_hi = lax.bitcast_convert_type(
        ci & jnp.int32(-65536), f32
    )
    h = lax.dot_general(
        e_lo,
        w[:, : _D // 2],
        (((1,), (1,)), ((), ())),
        preferred_element_type=f32,
    ) + lax.dot_general(
        e_hi,
        w[:, _D // 2:_D],
        (((1,), (1,)), ((), ())),
        preferred_element_type=f32,
    ) + lax.dot_general(
        taT.astype(bf16),
        w[:, _D:].astype(bf16),
        (((0,), (1,)), ((), ())),
        preferred_element_type=f32,
    )
    h = jnp.tanh(h + b_ref[...])

    pidx = jnp.clip(vo - vof, 0, _MAX_LEN - 1)
    lane = lax.broadcasted_iota(jnp.int32, (_TOK, _MAX_LEN), 1)
    ohpe = (lane == pidx).astype(bf16)
    h = h + lax.dot_general(
        ohpe, pe_ref[...], (((1,), (0,)), ((), ())),
        preferred_element_type=f32,
    )

    pad = cid == _PAD_IDX
    fp = cidf == _PAD_IDX
    g = jnp.where(pad, pe511_ref[...], pepat_ref[...])
    h = h + jnp.where(fp, pe0_ref[...], g)

    tidr, vsegr = tri[3:4, :], tri[4:5, :]
    sub32 = lax.broadcasted_iota(jnp.int32, (32, _TOK), 0)
    ohtvT = (sub32 == tidr).astype(bf16) + (
        sub32 == (vsegr + 16)
    ).astype(bf16)
    h = h + lax.dot_general(
        ohtvT, tv_ref[...].astype(bf16), (((0,), (0,)), ((), ())),
        preferred_element_type=f32,
    )

    mu = jnp.mean(h, axis=1, keepdims=True)
    hc = h - mu
    var = jnp.mean(hc * hc, axis=1, keepdims=True)
    out_ref[...] = hc * lax.rsqrt(var + 1e-12) * gamma_ref[...] + beta_ref[...]


def _epilogue_specs():
    tok_spec = lambda shp: pl.BlockSpec(shp, lambda i: (i, 0))
    full_spec = lambda shp: pl.BlockSpec(shp, lambda i: (0, 0))
    in_specs = [
        tok_spec((_TOK, _D // 2)),
        tok_spec((_TOK, 8)),
        pl.BlockSpec((8, _TOK), lambda i: (0, i)),
    ] + [
        full_spec((_D, _D + 2 * _T)),
        full_spec((1, _D)),
        full_spec((1, _D)),
        full_spec((1, _D)),
        full_spec((_T, 1)),
        full_spec((_T, 1)),
        full_spec((_T, 1)),
        full_spec((_T, 1)),
        full_spec((32, _D)),
        full_spec((_MAX_LEN, _D)),
        full_spec((_TOK, _D)),
        full_spec((1, _D)),
        full_spec((1, _D)),
    ]
    out_spec = tok_spec((_TOK, _D))
    return in_specs, out_spec


def _epilogue_call(args, interpret=False):
    in_specs, out_spec = _epilogue_specs()
    return pl.pallas_call(
        _epilogue_body,
        grid=(_N // _TOK,),
        in_specs=in_specs,
        out_specs=out_spec,
        out_shape=jax.ShapeDtypeStruct((_N, _D), jnp.float32),
        interpret=interpret,
    )(*args)


def kernel(concept_ids, type_ids, time_stamps, ages, visit_orders,
           visit_segments, concept_table, type_table, time_w, time_phi,
           age_w, age_phi, visit_table, W, b, gamma, beta):
    ids_flat = concept_ids.astype(jnp.int32).reshape(_N)
    tu = lax.bitcast_convert_type(
        concept_table.astype(jnp.bfloat16), jnp.uint16
    ).astype(jnp.uint32)
    table_p = lax.bitcast_convert_type(
        tu[:, : _D // 2] | (tu[:, _D // 2:] << 16), jnp.int32
    )
    conc = _sc_gather_fn(concept_table.shape[0], _D // 2, _N, jnp.int32)(
        table_p, ids_flat
    )

    col = lambda x: x.astype(jnp.int32).reshape(_N)
    first_col = lambda x: jnp.broadcast_to(
        x[:, :1], (_B, _S)
    ).astype(jnp.int32).reshape(_N)
    tsp = jnp.concatenate([time_stamps[:, :1], time_stamps[:, :-1]], axis=1)
    zeros = jnp.zeros((_N,), jnp.int32)
    ints = jnp.stack(
        [
            col(visit_orders), first_col(visit_orders),
            col(concept_ids), first_col(concept_ids),
            zeros, zeros, zeros, zeros,
        ],
        axis=1,
    )
    tsd = jnp.stack(
        [
            col(time_stamps), col(tsp), col(ages),
            col(type_ids), col(visit_segments),
            zeros, zeros, zeros,
        ],
        axis=0,
    )

    tv_table = jnp.zeros((32, _D), jnp.float32)
    tv_table = tv_table.at[0:9].set(type_table)
    tv_table = tv_table.at[16:19].set(visit_table)
    pe_np = _make_pe(_MAX_LEN, _D)
    pe_bf = jnp.asarray(pe_np, dtype=jnp.bfloat16)
    pepat = jnp.asarray(np.tile(pe_np, (_TOK // _MAX_LEN, 1)))
    pe0 = jnp.asarray(pe_np[0:1])
    pe511 = jnp.asarray(pe_np[_MAX_LEN - 1:_MAX_LEN])

    args = (
        conc, ints, tsd,
        W, b.reshape(1, _D), gamma.reshape(1, _D), beta.reshape(1, _D),
        time_w.reshape(_T, 1), time_phi.reshape(_T, 1),
        age_w.reshape(_T, 1), age_phi.reshape(_T, 1),
        tv_table, pe_bf, pepat, pe0, pe511,
    )
    out = _epilogue_call(args)
    return out.reshape(_B, _S, _D)


# speedup vs baseline: 1.2864x; 1.2864x over previous
import functools
import math

import jax
import jax.numpy as jnp
import numpy as np
from jax import lax
from jax.experimental import pallas as pl
from jax.experimental.pallas import tpu as pltpu
from jax.experimental.pallas import tpu_sc as plsc

_B, _S = 1024, 512
_D, _T = 128, 16
_MAX_LEN = 512
_PAD_IDX = 1
_N = _B * _S


def _make_pe(max_len, d):
    position = np.arange(max_len, dtype=np.float32)[:, None]
    div_term = np.exp(
        np.arange(0, d, 2, dtype=np.float32) * -(math.log(10000.0) / d)
    )
    pe = np.zeros((max_len, d), dtype=np.float32)
    pe[:, 0::2] = np.sin(position * div_term)
    pe[:, 1::2] = np.cos(position * div_term)
    return pe


_SC_CHUNK = 128
_SC_NBUF = 4


def _sc_gather_fn(V, D, N, dtype=jnp.float32):
    info = plsc.get_sparse_core_info()
    nc, ns = info.num_cores, info.num_subcores
    nw = nc * ns
    n_per_w = N // nw
    n_super = n_per_w // (_SC_CHUNK * _SC_NBUF)
    assert n_per_w % (_SC_CHUNK * _SC_NBUF) == 0

    mesh = plsc.VectorSubcoreMesh(core_axis_name="c", subcore_axis_name="s")

    @functools.partial(
        pl.kernel,
        mesh=mesh,
        out_type=jax.ShapeDtypeStruct((N, D), dtype),
        scratch_types=[
            pltpu.VMEM((n_per_w,), jnp.int32),
            pltpu.VMEM((_SC_NBUF, _SC_CHUNK, D), dtype),
            pltpu.SemaphoreType.DMA,
            pltpu.SemaphoreType.DMA,
        ],
    )
    def k(table_hbm, idx_hbm, out_hbm, idx_v, rows_v, sem_g, sem_w):
        wid = lax.axis_index("s") * nc + lax.axis_index("c")
        base = wid * n_per_w
        pltpu.sync_copy(idx_hbm.at[pl.ds(base, n_per_w)], idx_v)

        def super_step(q, _):
            c0 = q * _SC_NBUF
            gathers = []
            for b in range(_SC_NBUF):
                gathers.append(pltpu.async_copy(
                    table_hbm.at[
                        idx_v.at[pl.ds((c0 + b) * _SC_CHUNK, _SC_CHUNK)]
                    ],
                    rows_v.at[b],
                    sem_g,
                ))
            for h in gathers:
                h.wait()
            writes = []
            for b in range(_SC_NBUF):
                writes.append(pltpu.async_copy(
                    rows_v.at[b],
                    out_hbm.at[pl.ds(base + (c0 + b) * _SC_CHUNK, _SC_CHUNK)],
                    sem_w,
                ))
            for h in writes:
                h.wait()
            return 0

        lax.fori_loop(0, n_super, super_step, 0, unroll=False)

    return k


_TOK = 8192


def _fast_sin(x):
    n = jnp.round(x * 0.15915494309189535)
    r = (x - n * 6.28125) - n * 0.0019353071795864769
    r2 = r * r
    p = 2.1470496156333067e-06
    p = p * r2 - 0.00019263169952706073
    p = p * r2 + 0.008308849931229644
    p = p * r2 - 0.16662401538302815
    p = p * r2 + 0.9999791148945345
    return p * r


def _epilogue_body(
    conc_ref,
    ints_ref,
    tsd_ref,
    w_ref,
    b_ref,
    gamma_ref,
    beta_ref,
    twc_ref,
    tphc_ref,
    awc_ref,
    aphc_ref,
    tv_ref,
    pe_ref,
    pepat_ref,
    pe0_ref,
    pe511_ref,
    out_ref,
):
    f32 = jnp.float32
    bf16 = jnp.bfloat16

    ints = ints_ref[...]
    vo, vof = ints[:, 0:1], ints[:, 1:2]
    cid, cidf = ints[:, 2:3], ints[:, 3:4]

    tri = tsd_ref[...]
    dd = (tri[0:1, :] - tri[1:2, :]).astype(f32)
    agr = tri[2:3, :].astype(f32)
    tfT = _fast_sin(dd * twc_ref[...] + tphc_ref[...])
    afT = _fast_sin(agr * awc_ref[...] + aphc_ref[...])
    taT = jnp.concatenate([tfT, afT], axis=0)

    w = w_ref[...]
    h = lax.dot_general(
        conc_ref[...],
        w[:, :_D],
        (((1,), (1,)), ((), ())),
        preferred_element_type=f32,
    ) + lax.dot_general(
        taT.astype(bf16),
        w[:, _D:].astype(bf16),
        (((0,), (1,)), ((), ())),
        preferred_element_type=f32,
    )
    h = jnp.tanh(h + b_ref[...])

    pidx = jnp.clip(vo - vof, 0, _MAX_LEN - 1)
    lane = lax.broadcasted_iota(jnp.int32, (_TOK, _MAX_LEN), 1)
    ohpe = (lane == pidx).astype(bf16)
    h = h + lax.dot_general(
        ohpe, pe_ref[...], (((1,), (0,)), ((), ())),
        preferred_element_type=f32,
    )

    pad = cid == _PAD_IDX
    fp = cidf == _PAD_IDX
    g = jnp.where(pad, pe511_ref[...], pepat_ref[...])
    h = h + jnp.where(fp, pe0_ref[...], g)

    tidr, vsegr = tri[3:4, :], tri[4:5, :]
    sub32 = lax.broadcasted_iota(jnp.int32, (32, _TOK), 0)
    ohtvT = (sub32 == tidr).astype(bf16) + (
        sub32 == (vsegr + 16)
    ).astype(bf16)
    h = h + lax.dot_general(
        ohtvT, tv_ref[...].astype(bf16), (((0,), (0,)), ((), ())),
        preferred_element_type=f32,
    )

    mu = jnp.mean(h, axis=1, keepdims=True)
    hc = h - mu
    var = jnp.mean(hc * hc, axis=1, keepdims=True)
    out_ref[...] = hc * lax.rsqrt(var + 1e-12) * gamma_ref[...] + beta_ref[...]


def _epilogue_specs():
    tok_spec = lambda shp: pl.BlockSpec(shp, lambda i: (i, 0))
    full_spec = lambda shp: pl.BlockSpec(shp, lambda i: (0, 0))
    in_specs = [
        tok_spec((_TOK, _D)),
        tok_spec((_TOK, 8)),
        pl.BlockSpec((8, _TOK), lambda i: (0, i)),
    ] + [
        full_spec((_D, _D + 2 * _T)),
        full_spec((1, _D)),
        full_spec((1, _D)),
        full_spec((1, _D)),
        full_spec((_T, 1)),
        full_spec((_T, 1)),
        full_spec((_T, 1)),
        full_spec((_T, 1)),
        full_spec((32, _D)),
        full_spec((_MAX_LEN, _D)),
        full_spec((_TOK, _D)),
        full_spec((1, _D)),
        full_spec((1, _D)),
    ]
    out_spec = tok_spec((_TOK, _D))
    return in_specs, out_spec


def _epilogue_call(args, interpret=False):
    in_specs, out_spec = _epilogue_specs()
    return pl.pallas_call(
        _epilogue_body,
        grid=(_N // _TOK,),
        in_specs=in_specs,
        out_specs=out_spec,
        out_shape=jax.ShapeDtypeStruct((_N, _D), jnp.float32),
        interpret=interpret,
    )(*args)


def kernel(concept_ids, type_ids, time_stamps, ages, visit_orders,
           visit_segments, concept_table, type_table, time_w, time_phi,
           age_w, age_phi, visit_table, W, b, gamma, beta):
    ids_flat = concept_ids.astype(jnp.int32).reshape(_N)
    conc = _sc_gather_fn(concept_table.shape[0], _D, _N)(
        concept_table, ids_flat
    )

    col = lambda x: x.astype(jnp.int32).reshape(_N)
    first_col = lambda x: jnp.broadcast_to(
        x[:, :1], (_B, _S)
    ).astype(jnp.int32).reshape(_N)
    tsp = jnp.concatenate([time_stamps[:, :1], time_stamps[:, :-1]], axis=1)
    zeros = jnp.zeros((_N,), jnp.int32)
    ints = jnp.stack(
        [
            col(visit_orders), first_col(visit_orders),
            col(concept_ids), first_col(concept_ids),
            zeros, zeros, zeros, zeros,
        ],
        axis=1,
    )
    tsd = jnp.stack(
        [
            col(time_stamps), col(tsp), col(ages),
            col(type_ids), col(visit_segments),
            zeros, zeros, zeros,
        ],
        axis=0,
    )

    tv_table = jnp.zeros((32, _D), jnp.float32)
    tv_table = tv_table.at[0:9].set(type_table)
    tv_table = tv_table.at[16:19].set(visit_table)
    pe_np = _make_pe(_MAX_LEN, _D)
    pe_bf = jnp.asarray(pe_np, dtype=jnp.bfloat16)
    pepat = jnp.asarray(np.tile(pe_np, (_TOK // _MAX_LEN, 1)))
    pe0 = jnp.asarray(pe_np[0:1])
    pe511 = jnp.asarray(pe_np[_MAX_LEN - 1:_MAX_LEN])

    args = (
        conc, ints, tsd,
        W, b.reshape(1, _D), gamma.reshape(1, _D), beta.reshape(1, _D),
        time_w.reshape(_T, 1), time_phi.reshape(_T, 1),
        age_w.reshape(_T, 1), age_phi.reshape(_T, 1),
        tv_table, pe_bf, pepat, pe0, pe511,
    )
    out = _epilogue_call(args)
    return out.reshape(_B, _S, _D)
